# baseline (speedup 1.0000x reference)
import functools

import jax
import jax.numpy as jnp
from jax import lax
from jax.experimental import pallas as pl
from jax.experimental.pallas import tpu as pltpu
from jax.experimental.pallas import tpu_sc as plsc

D = 64
B = 4096
HIST = 50
NC, NS, L = 2, 16, 16
NW = NC * NS
BPW = B // NW

_mesh = plsc.VectorSubcoreMesh(core_axis_name="c", subcore_axis_name="s")


@functools.partial(
    pl.kernel,
    mesh=_mesh,
    out_type=jax.ShapeDtypeStruct((B,), jnp.float32),
    scratch_types=(
        [
            pltpu.VMEM((BPW, HIST), jnp.int32),
            pltpu.VMEM((HIST, BPW), jnp.int32),
            pltpu.VMEM((BPW,), jnp.int32),
            pltpu.VMEM((BPW, D), jnp.float32),
            pltpu.VMEM((BPW, D), jnp.float32),
            pltpu.VMEM((BPW,), jnp.float32),
            pltpu.VMEM((BPW,), jnp.float32),
        ]
        + [pltpu.SemaphoreType.DMA, pltpu.SemaphoreType.DMA]
    ),
    compiler_params=pltpu.CompilerParams(use_tc_tiling_on_sc=False),
)
def _bprmf_sc(seq_hbm, tgt_hbm, table_hbm, out_hbm,
              st, st_t, tgt_idx, tgt_rows, acc, wbuf, out_buf,
              gsem, tsem):
    wid = lax.axis_index("s") * NC + lax.axis_index("c")
    base = wid * BPW

    lane = lax.iota(jnp.int32, L)
    zero = jnp.zeros((L,), jnp.float32)
    one = jnp.ones((L,), jnp.float32)

    pltpu.sync_copy(seq_hbm.at[pl.ds(base, BPW)], st)
    pltpu.sync_copy(tgt_hbm.at[wid], tgt_idx)

    pltpu.async_copy(table_hbm.at[tgt_idx], tgt_rows, tsem)

    def _zbody(b, carry):
        for k in range(D // L):
            acc[b, pl.ds(k * L, L)] = zero
        return carry
    lax.fori_loop(0, BPW, _zbody, 0, unroll=8)

    def _allreduce_sum(v):
        for k in (8, 4, 2, 1):
            v = v + v.at[lane ^ k].get(mode="promise_in_bounds")
        return v

    descs = []
    for l0, wl in ((0, L), (L, L), (2 * L, L), (HIST - L, 2)):
        def _tbody(g, carry, l0=l0, wl=wl):
            b0 = g * L
            v = [st[b0 + i, pl.ds(l0, L)] for i in range(L)]
            for k in (8, 4, 2, 1):
                v = [jnp.where((lane & k) == (i & k), v[i],
                               v[i ^ k].at[lane ^ k].get(
                                   mode="promise_in_bounds"))
                     for i in range(L)]
            for j in range(L - wl, L):
                st_t[l0 + j, pl.ds(b0, L)] = v[j]
            return carry
        lax.fori_loop(0, BPW // L, _tbody, 0)
        for j in range(L - wl, L):
            descs.append(pltpu.async_copy(
                table_hbm.at[st_t.at[l0 + j]], acc, gsem, add=True))

    tail_lane = 3 * L - (HIST - L)
    def _cbody(t, carry):
        res = zero
        for j in range(L):
            b = t * L + j
            p = zero
            for k in range(3):
                s = st[b, pl.ds(k * L, L)]
                p = p + jnp.where(s != 0, one, zero)
            s = st[b, pl.ds(HIST - L, L)]
            p = p + jnp.where((s != 0) & (lane >= tail_lane), one, zero)
            res = jnp.where(lane == j, _allreduce_sum(p), res)
        wbuf[pl.ds(t * L, L)] = res
        return carry
    lax.fori_loop(0, BPW // L, _cbody, 0)

    pltpu.make_async_copy(table_hbm.at[tgt_idx], tgt_rows, tsem).wait()
    for d in descs:
        d.wait()

    def _obody(t, carry):
        res = zero
        for j in range(L):
            b = t * L + j
            dotv = zero
            for k in range(D // L):
                dotv = dotv + (acc[b, pl.ds(k * L, L)]
                               * tgt_rows[b, pl.ds(k * L, L)])
            pred_v = _allreduce_sum(dotv)
            res = jnp.where(lane == j, pred_v, res)
        w = wbuf[pl.ds(t * L, L)]
        out_buf[pl.ds(t * L, L)] = res / w
        return carry
    lax.fori_loop(0, BPW // L, _obody, 0)

    pltpu.sync_copy(out_buf, out_hbm.at[pl.ds(base, BPW)])


def kernel(seq, target, embed_weight):
    tgt_w = target.astype(jnp.int32).reshape(NW, BPW)
    return _bprmf_sc(seq.astype(jnp.int32), tgt_w, embed_weight)
